# baseline (speedup 1.0000x reference)
import functools
import jax
import jax.numpy as jnp
from jax.experimental import pallas as pl
from jax.experimental.pallas import tpu as pltpu

N = 10000
D = 512
K = 2048

_RB = 128
_NRB = (N + _RB - 1) // _RB
_WINDOWS = [(0, 16), (16, 16), (32, 16), (48, 16), (64, 15)]


def _degrees_body(a_ref, o_ref):
    x = a_ref[...]
    tot = None
    for (c0, nch) in _WINDOWS:
        p = None
        for t in range(nch):
            lo = (c0 + t) * 128
            hi = lo + 128
            if hi <= N:
                c = x[:, lo:hi]
            else:
                c = jnp.concatenate(
                    [x[:, lo:N], jnp.zeros((_RB, hi - N), jnp.float32)], axis=1)
            p = c if p is None else p + c
        t_p = p.T
        acc = t_p[0:8, :]
        for k in range(1, 16):
            acc = acc + t_p[8 * k:8 * k + 8, :]
        acc = acc[0:4, :] + acc[4:8, :]
        acc = acc[0:2, :] + acc[2:4, :]
        acc = acc[0:1, :] + acc[1:2, :]
        tot = acc if tot is None else tot + acc
    o_ref[...] = tot[None]


def _degrees(adjacency):
    out = pl.pallas_call(
        _degrees_body,
        grid=(_NRB,),
        in_specs=[pl.BlockSpec((_RB, N), lambda i: (i, 0))],
        out_specs=pl.BlockSpec((1, 1, _RB), lambda i: (i, 0, 0)),
        out_shape=jax.ShapeDtypeStruct((_NRB, 1, _RB), jnp.float32),
    )(adjacency)
    return out.reshape(_NRB * _RB)[:N]


def kernel(x, adjacency):
    degrees = _degrees(adjacency)
    _, idx = jax.lax.top_k(degrees, K)
    x_pooled = x[idx]
    adj_pooled = adjacency[idx][:, idx]
    return (x_pooled, adj_pooled)
